# baseline (speedup 1.0000x reference)
import functools

import jax
import jax.numpy as jnp
from jax import lax
from jax.experimental import pallas as pl
from jax.experimental.pallas import tpu as pltpu
from jax.experimental.pallas import tpu_sc as plsc


def _iota16():
    return lax.iota(jnp.int32, 16)


@functools.lru_cache(maxsize=None)
def _gather_call(s, b, v, d):
    info = plsc.get_sparse_core_info()
    nw = info.num_cores * info.num_subcores
    bc = b // nw

    mesh = plsc.VectorSubcoreMesh(core_axis_name="c", subcore_axis_name="s")

    def body(pairs_hbm, ids_hbm, out_hbm, idx_v, q_v, g_v, a_v,
             gs0, gs1, ws0, ws1):
        cid = lax.axis_index("c")
        sid = lax.axis_index("s")
        wid = sid * info.num_cores + cid
        b_off = pl.multiple_of(wid * bc, 128)
        gsem = (gs0, gs1)
        wsem = (ws0, ws1)
        pltpu.sync_copy(ids_hbm.at[:, pl.ds(b_off, bc)], idx_v)

        def fire(ci, slot):
            for g in range(bc // 16):
                raw = idx_v[ci, pl.ds(g * 16, 16)]
                q_v[slot, pl.ds(g * 16, 16)] = lax.shift_right_logical(raw, 1)
            pltpu.async_copy(pairs_hbm.at[q_v.at[slot]], g_v.at[slot],
                             gsem[slot])

        def wait_gather(slot):
            pltpu.make_async_copy(pairs_hbm.at[q_v.at[slot]], g_v.at[slot],
                                  gsem[slot]).wait()

        def wait_write(slot):
            pltpu.make_async_copy(
                a_v.at[slot], out_hbm.at[0, pl.ds(0, bc), :],
                wsem[slot]).wait()

        fire(0, 0)

        ntrip = s // 2

        def trip(t, carry):
            for k in range(2):
                ci = 2 * t + k

                @pl.when(ci + 1 < s)
                def _(ci=ci, k=k):
                    fire(ci + 1, 1 - k)
                wait_gather(k)

                @pl.when(ci >= 2)
                def _(k=k):
                    wait_write(k)

                for g in range(bc // 16):
                    jrow = _iota16() + g * 16
                    raw = idx_v[ci, pl.ds(g * 16, 16)]
                    off = lax.shift_left(lax.bitwise_and(raw, 1), 6)

                    def crow(c8, c2, jrow=jrow, off=off, k=k):
                        c0 = c8 * 8
                        for dc in range(8):
                            c = c0 + dc
                            vals = plsc.load_gather(g_v.at[k], [jrow, off + c])
                            plsc.store_scatter(
                                a_v.at[k],
                                [jrow, jnp.full((16,), c, jnp.int32)], vals)
                        return c2

                    lax.fori_loop(0, d // 8, crow, 0)

                pltpu.async_copy(
                    a_v.at[k], out_hbm.at[ci, pl.ds(b_off, bc), :], wsem[k])
            return carry

        lax.fori_loop(0, ntrip, trip, 0)
        wait_write(0)
        wait_write(1)

    return pl.kernel(
        body,
        out_type=jax.ShapeDtypeStruct((s, b, 2 * d), jnp.float32),
        mesh=mesh,
        scratch_types=(
            pltpu.VMEM((s, bc), jnp.int32),
            pltpu.VMEM((2, bc), jnp.int32),
            pltpu.VMEM((2, bc, 2 * d), jnp.float32),
            pltpu.VMEM((2, bc, 2 * d), jnp.float32),
            pltpu.SemaphoreType.DMA, pltpu.SemaphoreType.DMA,
            pltpu.SemaphoreType.DMA, pltpu.SemaphoreType.DMA,
        ),
        compiler_params=pltpu.CompilerParams(
            use_tc_tiling_on_sc=True, needs_layout_passes=False),
    )


def kernel(input_ids, table):
    s, b = input_ids.shape
    v, d = table.shape
    pairs = table.reshape(v // 2, 2 * d)
    out = _gather_call(s, b, v, d)(pairs, input_ids)
    return out[:, :, :d]


# speedup vs baseline: 2.6760x; 2.6760x over previous
import functools

import jax
import jax.numpy as jnp
from jax import lax
from jax.experimental import pallas as pl
from jax.experimental.pallas import tpu as pltpu
from jax.experimental.pallas import tpu_sc as plsc


@functools.lru_cache(maxsize=None)
def _gather_call(s, b, v, d):
    info = plsc.get_sparse_core_info()
    nw = info.num_cores * info.num_subcores
    bc = b // nw
    nch = s // 2

    mesh = plsc.VectorSubcoreMesh(core_axis_name="c", subcore_axis_name="s")

    def body(table_hbm, ids_hbm, out_hbm, idx_v, g_v, gs0, gs1, ws0, ws1):
        cid = lax.axis_index("c")
        sid = lax.axis_index("s")
        wid = sid * info.num_cores + cid
        b_off = pl.multiple_of(wid * bc, 128)
        gsem = (gs0, gs1)
        wsem = (ws0, ws1)
        pltpu.sync_copy(ids_hbm.at[:, pl.ds(b_off, bc)], idx_v)

        def fire(ci, slot):
            for h in range(2):
                pltpu.async_copy(table_hbm.at[idx_v.at[2 * ci + h]],
                                 g_v.at[slot, h], gsem[slot])

        def wait_gather(slot):
            for h in range(2):
                pltpu.make_async_copy(table_hbm.at[idx_v.at[0]],
                                      g_v.at[slot, h], gsem[slot]).wait()

        def wait_write(slot):
            pltpu.make_async_copy(
                g_v.at[slot],
                out_hbm.at[pl.ds(0, 2), pl.ds(0, bc), :], wsem[slot]).wait()

        fire(0, 0)

        ntrip = nch // 2

        def trip(t, carry):
            for k in range(2):
                ci = 2 * t + k

                @pl.when(ci + 1 < nch)
                def _(ci=ci, k=k):
                    @pl.when(ci >= 1)
                    def _():
                        wait_write(1 - k)
                    fire(ci + 1, 1 - k)
                wait_gather(k)
                s_off = pl.multiple_of(2 * ci, 2)
                pltpu.async_copy(
                    g_v.at[k],
                    out_hbm.at[pl.ds(s_off, 2), pl.ds(b_off, bc), :], wsem[k])
            return carry

        lax.fori_loop(0, ntrip, trip, 0)
        wait_write(0)
        wait_write(1)

    return pl.kernel(
        body,
        out_type=jax.ShapeDtypeStruct((s, b, 2 * d), jnp.float32),
        mesh=mesh,
        scratch_types=(
            pltpu.VMEM((s, bc), jnp.int32),
            pltpu.VMEM((2, 2, bc, 2 * d), jnp.float32),
            pltpu.SemaphoreType.DMA, pltpu.SemaphoreType.DMA,
            pltpu.SemaphoreType.DMA, pltpu.SemaphoreType.DMA,
        ),
        compiler_params=pltpu.CompilerParams(
            use_tc_tiling_on_sc=True, needs_layout_passes=False),
    )


def kernel(input_ids, table):
    s, b = input_ids.shape
    v, d = table.shape
    padded = jnp.pad(table, ((0, 0), (0, 128 - d)))
    out = _gather_call(s, b, v, d)(padded, input_ids)
    return out[:, :, :d]
